# baseline (speedup 1.0000x reference)
import functools

import jax
import jax.numpy as jnp
from jax import lax
from jax.experimental import pallas as pl
from jax.experimental.pallas import tpu as pltpu
from jax.experimental.pallas import tpu_sc as plsc

ROWS = 64
N = 32768
L = 16
NB = N // L
U = 8
NG = NB // U
T_NEWTON = 10
C = 8192
NEG = -3e38

_NC = 2
_NS = 16
NW = _NC * _NS
ROWS_PER = ROWS // NW


def _tree_max8(c):
    t01 = jnp.maximum(c[0], c[1])
    t23 = jnp.maximum(c[2], c[3])
    t45 = jnp.maximum(c[4], c[5])
    t67 = jnp.maximum(c[6], c[7])
    return jnp.maximum(jnp.maximum(t01, t23), jnp.maximum(t45, t67))


def _fused_max_collect(row_v, cand_v):
    g0 = [row_v[pl.ds(j * L, L)] for j in range(U)]
    m0 = _tree_max8(g0)
    w = jnp.broadcast_to(jnp.max(m0), (L,)) - 1.0

    def body(g, carry):
        m, t0, t1, off = carry
        base = g * (U * L)
        c = [row_v[pl.ds(base + j * L, L)] for j in range(U)]
        for j in range(U):
            v = c[j]
            msk = v > t0
            cnt = plsc.all_reduce_population_count(msk)[0]
            cand_v[pl.ds(off, L)] = v
            off = off + jnp.where(jnp.logical_and(cnt > 0, off < C - L),
                                  jnp.int32(L), jnp.int32(0))
        m_new = jnp.maximum(m, _tree_max8(c))
        nt = jnp.broadcast_to(jnp.max(m_new), (L,)) - 1.0
        return (m_new, t1, nt, off)

    m, _, _, off = lax.fori_loop(0, NG, body, (m0, w, w, jnp.int32(0)))
    return jnp.max(m), off


def _shrink(cand_v, vals_v, nb_c, thr):
    def body(i, off2):
        v = cand_v[pl.ds(i * L, L)]
        msk = v > thr
        cnt = plsc.all_reduce_population_count(msk)[0]
        plsc.store_compressed(vals_v.at[pl.ds(off2, L)], v, mask=msk)
        return off2 + cnt
    k1 = lax.fori_loop(0, nb_c, body, jnp.int32(0))
    vals_v[pl.ds(k1, L)] = jnp.full((L,), NEG, jnp.float32)
    return k1


def _row_sparsemax(row_v, cand_v, vals_v):
    mx, off = _fused_max_collect(row_v, cand_v)
    thr = jnp.broadcast_to(mx, (L,)) - 1.0
    overflow = off >= C - L

    k1 = _shrink(cand_v, vals_v, off >> 4, thr)
    nv = (k1 + (L - 1)) >> 4

    def _sk_pass(ref, n_chunks, tau):
        def b(i, sk):
            sv, kv = sk
            v = ref[pl.ds(i * L, L)]
            msk = v > tau
            sv = sv + jnp.where(msk, v, jnp.float32(0))
            kv = kv + msk.astype(jnp.int32)
            return (sv, kv)
        return lax.fori_loop(
            0, n_chunks, b,
            (jnp.zeros((L,), jnp.float32), jnp.zeros((L,), jnp.int32)))

    def newton_body(t, tau):
        sv, kv = lax.cond(
            overflow,
            lambda tt: _sk_pass(row_v, NB, tt),
            lambda tt: _sk_pass(vals_v, nv, tt),
            tau)
        s = jnp.sum(sv)
        kf = jnp.sum(kv.astype(jnp.float32))
        kfv = jnp.maximum(jnp.broadcast_to(kf, (L,)), 1.0)
        tau_new = (jnp.broadcast_to(s, (L,)) - 1.0) / kfv
        return jnp.maximum(tau, tau_new)
    tau = lax.fori_loop(0, T_NEWTON, newton_body, thr)

    def out_body(g):
        base = g * (U * L)
        for j in range(U):
            sl = pl.ds(base + j * L, L)
            row_v[sl] = jnp.maximum(row_v[sl] - tau, jnp.float32(0))
    plsc.parallel_loop(0, NG, 1, unroll=2)(out_body)


def _body(x_hbm, out_hbm, row_a, row_b, cand_v, vals_v, sem_a, sem_b):
    wid = lax.axis_index("s") * _NC + lax.axis_index("c")
    r0 = wid * ROWS_PER
    r1 = r0 + 1
    in_a = pltpu.async_copy(x_hbm.at[r0], row_a, sem_a)
    in_b = pltpu.async_copy(x_hbm.at[r1], row_b, sem_b)
    in_a.wait()
    _row_sparsemax(row_a, cand_v, vals_v)
    out_a = pltpu.async_copy(row_a, out_hbm.at[r0], sem_a)
    in_b.wait()
    _row_sparsemax(row_b, cand_v, vals_v)
    out_b = pltpu.async_copy(row_b, out_hbm.at[r1], sem_b)
    out_a.wait()
    out_b.wait()


@jax.jit
def kernel(input):
    mesh = plsc.VectorSubcoreMesh(core_axis_name="c", subcore_axis_name="s")
    f = pl.kernel(
        _body,
        out_type=jax.ShapeDtypeStruct((ROWS, N), jnp.float32),
        mesh=mesh,
        scratch_types=[
            pltpu.VMEM((N,), jnp.float32),
            pltpu.VMEM((N,), jnp.float32),
            pltpu.VMEM((C,), jnp.float32),
            pltpu.VMEM((C,), jnp.float32),
            pltpu.SemaphoreType.DMA,
            pltpu.SemaphoreType.DMA,
        ],
        compiler_params=pltpu.CompilerParams(needs_layout_passes=False),
    )
    return f(input)


# speedup vs baseline: 1.0240x; 1.0240x over previous
import functools

import jax
import jax.numpy as jnp
from jax import lax
from jax.experimental import pallas as pl
from jax.experimental.pallas import tpu as pltpu
from jax.experimental.pallas import tpu_sc as plsc

ROWS = 64
N = 32768
L = 16
NB = N // L
U = 8
NG = NB // U
T_NEWTON = 10
C = 16384
NEG = -3e38

_NC = 2
_NS = 16
NW = _NC * _NS
ROWS_PER = ROWS // NW


def _tree_max8(c):
    t01 = jnp.maximum(c[0], c[1])
    t23 = jnp.maximum(c[2], c[3])
    t45 = jnp.maximum(c[4], c[5])
    t67 = jnp.maximum(c[6], c[7])
    return jnp.maximum(jnp.maximum(t01, t23), jnp.maximum(t45, t67))


def _fused_max_collect(row_v, cand_v):
    g0 = [row_v[pl.ds(j * L, L)] for j in range(U)]
    m0 = _tree_max8(g0)
    w = jnp.broadcast_to(jnp.max(m0), (L,)) - 1.0

    def body(g, carry):
        m, t0, t1, off = carry
        base = g * (U * L)
        c = [row_v[pl.ds(base + j * L, L)] for j in range(U)]
        for j in range(0, U, 2):
            msk = jnp.logical_or(c[j] > t0, c[j + 1] > t0)
            cnt = plsc.all_reduce_population_count(msk)[0]
            cand_v[pl.ds(off, L)] = c[j]
            cand_v[pl.ds(off + L, L)] = c[j + 1]
            off = off + jnp.where(jnp.logical_and(cnt > 0, off < C - 4 * L),
                                  jnp.int32(2 * L), jnp.int32(0))
        m_new = jnp.maximum(m, _tree_max8(c))
        nt = jnp.broadcast_to(jnp.max(m_new), (L,)) - 1.0
        return (m_new, t1, nt, off)

    m, _, _, off = lax.fori_loop(0, NG, body, (m0, w, w, jnp.int32(0)))
    return jnp.max(m), off


def _shrink(cand_v, vals_v, nb_c, thr):
    def body(i, off2):
        v = cand_v[pl.ds(i * L, L)]
        msk = v > thr
        cnt = plsc.all_reduce_population_count(msk)[0]
        plsc.store_compressed(vals_v.at[pl.ds(off2, L)], v, mask=msk)
        return off2 + cnt
    k1 = lax.fori_loop(0, nb_c, body, jnp.int32(0))
    vals_v[pl.ds(k1, L)] = jnp.full((L,), NEG, jnp.float32)
    return k1


def _row_sparsemax(row_v, cand_v, vals_v):
    mx, off = _fused_max_collect(row_v, cand_v)
    thr = jnp.broadcast_to(mx, (L,)) - 1.0
    overflow = off >= C - 2 * L

    k1 = _shrink(cand_v, vals_v, off >> 4, thr)
    nv = (k1 + (L - 1)) >> 4

    def _sk_pass(ref, n_chunks, tau):
        def b(i, sk):
            sv, kv = sk
            v = ref[pl.ds(i * L, L)]
            msk = v > tau
            sv = sv + jnp.where(msk, v, jnp.float32(0))
            kv = kv + msk.astype(jnp.int32)
            return (sv, kv)
        return lax.fori_loop(
            0, n_chunks, b,
            (jnp.zeros((L,), jnp.float32), jnp.zeros((L,), jnp.int32)))

    def newton_body(t, tau):
        sv, kv = lax.cond(
            overflow,
            lambda tt: _sk_pass(row_v, NB, tt),
            lambda tt: _sk_pass(vals_v, nv, tt),
            tau)
        s = jnp.sum(sv)
        kf = jnp.sum(kv.astype(jnp.float32))
        kfv = jnp.maximum(jnp.broadcast_to(kf, (L,)), 1.0)
        tau_new = (jnp.broadcast_to(s, (L,)) - 1.0) / kfv
        return jnp.maximum(tau, tau_new)
    tau = lax.fori_loop(0, T_NEWTON, newton_body, thr)

    def out_body(g):
        base = g * (U * L)
        for j in range(U):
            sl = pl.ds(base + j * L, L)
            row_v[sl] = jnp.maximum(row_v[sl] - tau, jnp.float32(0))
    plsc.parallel_loop(0, NG, 1, unroll=2)(out_body)


def _body(x_hbm, out_hbm, row_a, row_b, cand_v, vals_v, sem_a, sem_b):
    wid = lax.axis_index("s") * _NC + lax.axis_index("c")
    r0 = wid * ROWS_PER
    r1 = r0 + 1
    in_a = pltpu.async_copy(x_hbm.at[r0], row_a, sem_a)
    in_b = pltpu.async_copy(x_hbm.at[r1], row_b, sem_b)
    in_a.wait()
    _row_sparsemax(row_a, cand_v, vals_v)
    out_a = pltpu.async_copy(row_a, out_hbm.at[r0], sem_a)
    in_b.wait()
    _row_sparsemax(row_b, cand_v, vals_v)
    out_b = pltpu.async_copy(row_b, out_hbm.at[r1], sem_b)
    out_a.wait()
    out_b.wait()


@jax.jit
def kernel(input):
    mesh = plsc.VectorSubcoreMesh(core_axis_name="c", subcore_axis_name="s")
    f = pl.kernel(
        _body,
        out_type=jax.ShapeDtypeStruct((ROWS, N), jnp.float32),
        mesh=mesh,
        scratch_types=[
            pltpu.VMEM((N,), jnp.float32),
            pltpu.VMEM((N,), jnp.float32),
            pltpu.VMEM((C,), jnp.float32),
            pltpu.VMEM((C,), jnp.float32),
            pltpu.SemaphoreType.DMA,
            pltpu.SemaphoreType.DMA,
        ],
        compiler_params=pltpu.CompilerParams(needs_layout_passes=False),
    )
    return f(input)


# speedup vs baseline: 1.0266x; 1.0025x over previous
import functools

import jax
import jax.numpy as jnp
from jax import lax
from jax.experimental import pallas as pl
from jax.experimental.pallas import tpu as pltpu
from jax.experimental.pallas import tpu_sc as plsc

ROWS = 64
N = 32768
L = 16
NB = N // L
U = 8
NG = NB // U
T_NEWTON = 10
C = 24576
GPI = 4
NEG = -3e38

_NC = 2
_NS = 16
NW = _NC * _NS
ROWS_PER = ROWS // NW


def _tree_max8(c):
    t01 = jnp.maximum(c[0], c[1])
    t23 = jnp.maximum(c[2], c[3])
    t45 = jnp.maximum(c[4], c[5])
    t67 = jnp.maximum(c[6], c[7])
    return jnp.maximum(jnp.maximum(t01, t23), jnp.maximum(t45, t67))


def _fused_max_collect(row_v, cand_v):
    g0 = [row_v[pl.ds(j * L, L)] for j in range(U)]
    m0 = _tree_max8(g0)
    w = jnp.broadcast_to(jnp.max(m0), (L,)) - 1.0

    def body(it, carry):
        m, t0, t1, off = carry
        m_new = m
        for gg in range(GPI):
            base = (it * GPI + gg) * (U * L)
            c = [row_v[pl.ds(base + j * L, L)] for j in range(U)]
            for j in range(0, U, 2):
                msk = jnp.logical_or(c[j] > t0, c[j + 1] > t0)
                cnt = plsc.all_reduce_population_count(msk)[0]
                cand_v[pl.ds(off, L)] = c[j]
                cand_v[pl.ds(off + L, L)] = c[j + 1]
                off = off + jnp.where(
                    jnp.logical_and(cnt > 0, off < C - 4 * L),
                    jnp.int32(2 * L), jnp.int32(0))
            m_new = jnp.maximum(m_new, _tree_max8(c))
        nt = jnp.broadcast_to(jnp.max(m_new), (L,)) - 1.0
        return (m_new, t1, nt, off)

    m, _, _, off = lax.fori_loop(0, NG // GPI, body, (m0, w, w, jnp.int32(0)))
    return jnp.max(m), off


def _shrink(cand_v, vals_v, nb_c, thr):
    def body(i, off2):
        v = cand_v[pl.ds(i * L, L)]
        msk = v > thr
        cnt = plsc.all_reduce_population_count(msk)[0]
        plsc.store_compressed(vals_v.at[pl.ds(off2, L)], v, mask=msk)
        return off2 + cnt
    k1 = lax.fori_loop(0, nb_c, body, jnp.int32(0))
    vals_v[pl.ds(k1, L)] = jnp.full((L,), NEG, jnp.float32)
    return k1


def _row_sparsemax(row_v, cand_v, vals_v):
    mx, off = _fused_max_collect(row_v, cand_v)
    thr = jnp.broadcast_to(mx, (L,)) - 1.0
    overflow = off >= C - 2 * L

    k1 = _shrink(cand_v, vals_v, off >> 4, thr)
    nv = (k1 + (L - 1)) >> 4

    def _sk_pass(ref, n_chunks, tau):
        def b(i, sk):
            sv, kv = sk
            v = ref[pl.ds(i * L, L)]
            msk = v > tau
            sv = sv + jnp.where(msk, v, jnp.float32(0))
            kv = kv + msk.astype(jnp.int32)
            return (sv, kv)
        return lax.fori_loop(
            0, n_chunks, b,
            (jnp.zeros((L,), jnp.float32), jnp.zeros((L,), jnp.int32)))

    def newton_body(t, tau):
        sv, kv = lax.cond(
            overflow,
            lambda tt: _sk_pass(row_v, NB, tt),
            lambda tt: _sk_pass(vals_v, nv, tt),
            tau)
        s = jnp.sum(sv)
        kf = jnp.sum(kv.astype(jnp.float32))
        kfv = jnp.maximum(jnp.broadcast_to(kf, (L,)), 1.0)
        tau_new = (jnp.broadcast_to(s, (L,)) - 1.0) / kfv
        return jnp.maximum(tau, tau_new)
    tau = lax.fori_loop(0, T_NEWTON, newton_body, thr)

    def out_body(g):
        base = g * (U * L)
        for j in range(U):
            sl = pl.ds(base + j * L, L)
            row_v[sl] = jnp.maximum(row_v[sl] - tau, jnp.float32(0))
    plsc.parallel_loop(0, NG, 1, unroll=2)(out_body)


def _body(x_hbm, out_hbm, row_a, row_b, cand_v, vals_v, sem_a, sem_b):
    wid = lax.axis_index("s") * _NC + lax.axis_index("c")
    r0 = wid * ROWS_PER
    r1 = r0 + 1
    in_a = pltpu.async_copy(x_hbm.at[r0], row_a, sem_a)
    in_b = pltpu.async_copy(x_hbm.at[r1], row_b, sem_b)
    in_a.wait()
    _row_sparsemax(row_a, cand_v, vals_v)
    out_a = pltpu.async_copy(row_a, out_hbm.at[r0], sem_a)
    in_b.wait()
    _row_sparsemax(row_b, cand_v, vals_v)
    out_b = pltpu.async_copy(row_b, out_hbm.at[r1], sem_b)
    out_a.wait()
    out_b.wait()


@jax.jit
def kernel(input):
    mesh = plsc.VectorSubcoreMesh(core_axis_name="c", subcore_axis_name="s")
    f = pl.kernel(
        _body,
        out_type=jax.ShapeDtypeStruct((ROWS, N), jnp.float32),
        mesh=mesh,
        scratch_types=[
            pltpu.VMEM((N,), jnp.float32),
            pltpu.VMEM((N,), jnp.float32),
            pltpu.VMEM((C,), jnp.float32),
            pltpu.VMEM((C,), jnp.float32),
            pltpu.SemaphoreType.DMA,
            pltpu.SemaphoreType.DMA,
        ],
        compiler_params=pltpu.CompilerParams(needs_layout_passes=False),
    )
    return f(input)
